# baseline (speedup 1.0000x reference)
import jax
import jax.numpy as jnp
from jax import lax
from jax.experimental import pallas as pl
from jax.experimental.pallas import tpu as pltpu
from jax.experimental.pallas import tpu_sc as plsc

NUM_USERS = 100000
NUM_ITEMS = 100000
EMBED_DIM = 64
BATCH = 16384

_info = plsc.get_sparse_core_info()
_NC, _NS, _L = _info.num_cores, _info.num_subcores, _info.num_lanes
_NW = _NC * _NS
_BPW = BATCH // _NW
_CHUNK = 256
_NCHUNK = _BPW // _CHUNK
_ROWS_PER_BLK = _L
_NBLK = _CHUNK // _ROWS_PER_BLK
_ROW = 2 * EMBED_DIM

_LW = 512
_TGRID = (NUM_USERS + _LW - 1) // _LW


def _tc_transpose_body(x_ref, y_ref):
    eye2 = jnp.concatenate(
        [jnp.eye(EMBED_DIM, dtype=jnp.float32)] * 2, axis=1)
    y_ref[...] = jax.lax.dot_general(
        x_ref[...], eye2, (((0,), (0,)), ((), ())),
        preferred_element_type=jnp.float32)


def _relayout(table_t):
    return pl.pallas_call(
        _tc_transpose_body,
        grid=(_TGRID,),
        in_specs=[pl.BlockSpec((EMBED_DIM, _LW), lambda i: (0, i))],
        out_specs=pl.BlockSpec((_LW, _ROW), lambda i: (i, 0)),
        out_shape=jax.ShapeDtypeStruct((NUM_USERS, _ROW), jnp.float32),
    )(table_t)


def _sc_body(uids_hbm, iids_hbm, utab_hbm, itab_hbm, out_hbm,
             uidx_v, iidx_v, urows_v, irows_v, out_v, part_v, sem_u, sem_i):
    wid = lax.axis_index("s") * _NC + lax.axis_index("c")
    base = wid * _BPW

    pltpu.sync_copy(uids_hbm.at[pl.ds(base, _BPW)], uidx_v)
    pltpu.sync_copy(iids_hbm.at[pl.ds(base, _BPW)], iidx_v)

    lane = lax.iota(jnp.int32, _L)

    for c in range(_NCHUNK):
        cu = pltpu.async_copy(
            utab_hbm.at[uidx_v.at[pl.ds(c * _CHUNK, _CHUNK)]], urows_v, sem_u)
        ci = pltpu.async_copy(
            itab_hbm.at[iidx_v.at[pl.ds(c * _CHUNK, _CHUNK)]], irows_v, sem_i)
        cu.wait()
        ci.wait()

        def blk(b, _):
            r0 = b * _ROWS_PER_BLK
            for k in range(_ROWS_PER_BLK):
                acc = (urows_v[r0 + k, pl.ds(0, _L)]
                       * irows_v[r0 + k, pl.ds(0, _L)])
                for d in range(1, EMBED_DIM // _L):
                    acc = acc + (urows_v[r0 + k, pl.ds(d * _L, _L)]
                                 * irows_v[r0 + k, pl.ds(d * _L, _L)])
                part_v[pl.ds(k * _L, _L)] = acc
            rowbase = lane * _L
            total = plsc.load_gather(part_v, [rowbase])
            for j in range(1, _L):
                total = total + plsc.load_gather(part_v, [rowbase + j])
            out_v[pl.ds(c * _CHUNK + r0, _L)] = 1.0 / (1.0 + jnp.exp(-total))
            return ()

        lax.fori_loop(0, _NBLK, blk, (), unroll=False)

    pltpu.sync_copy(out_v, out_hbm.at[pl.ds(base, _BPW)])


@jax.jit
def kernel(user_ids, item_ids, user_table, item_table):
    utab_p = _relayout(user_table.T)
    itab_p = _relayout(item_table.T)
    mesh = plsc.VectorSubcoreMesh(core_axis_name="c", subcore_axis_name="s")
    run = pl.kernel(
        _sc_body,
        out_type=jax.ShapeDtypeStruct((BATCH,), jnp.float32),
        mesh=mesh,
        scratch_types=[
            pltpu.VMEM((_BPW,), jnp.int32),
            pltpu.VMEM((_BPW,), jnp.int32),
            pltpu.VMEM((_CHUNK, _ROW), jnp.float32),
            pltpu.VMEM((_CHUNK, _ROW), jnp.float32),
            pltpu.VMEM((_BPW,), jnp.float32),
            pltpu.VMEM((_L * _L,), jnp.float32),
            pltpu.SemaphoreType.DMA,
            pltpu.SemaphoreType.DMA,
        ],
        compiler_params=pltpu.CompilerParams(
            needs_layout_passes=False, use_tc_tiling_on_sc=True),
    )
    return run(user_ids.astype(jnp.int32), item_ids.astype(jnp.int32),
               utab_p, itab_p)


# speedup vs baseline: 2.7651x; 2.7651x over previous
import jax
import jax.numpy as jnp
from jax import lax
from jax.experimental import pallas as pl
from jax.experimental.pallas import tpu as pltpu
from jax.experimental.pallas import tpu_sc as plsc

NUM_USERS = 100000
NUM_ITEMS = 100000
EMBED_DIM = 64
BATCH = 16384

_info = plsc.get_sparse_core_info()
_NC, _NS, _L = _info.num_cores, _info.num_subcores, _info.num_lanes
_NW = _NC * _NS
_BPW = BATCH // _NW
_CHUNK = 256
_NCHUNK = _BPW // _CHUNK
_ROWS_PER_BLK = _L
_NBLK = _CHUNK // _ROWS_PER_BLK
_ROW = 2 * EMBED_DIM

_LW = 4096
_TGRID = (NUM_USERS + _LW - 1) // _LW


def _tc_transpose_body(x_ref, y_ref):
    eye2 = jnp.concatenate(
        [jnp.eye(EMBED_DIM, dtype=jnp.float32)] * 2, axis=1)
    y_ref[...] = jax.lax.dot_general(
        x_ref[...], eye2, (((0,), (0,)), ((), ())),
        preferred_element_type=jnp.float32)


def _relayout(table_t):
    return pl.pallas_call(
        _tc_transpose_body,
        grid=(_TGRID,),
        in_specs=[pl.BlockSpec((EMBED_DIM, _LW), lambda i: (0, i))],
        out_specs=pl.BlockSpec((_LW, _ROW), lambda i: (i, 0)),
        out_shape=jax.ShapeDtypeStruct((NUM_USERS, _ROW), jnp.float32),
    )(table_t)


def _sc_body(uids_hbm, iids_hbm, utab_hbm, itab_hbm, out_hbm,
             uidx_v, iidx_v, urows_v, irows_v, out_v, part_v, sem_u, sem_i):
    wid = lax.axis_index("s") * _NC + lax.axis_index("c")
    base = wid * _BPW

    pltpu.sync_copy(uids_hbm.at[pl.ds(base, _BPW)], uidx_v)
    pltpu.sync_copy(iids_hbm.at[pl.ds(base, _BPW)], iidx_v)

    lane = lax.iota(jnp.int32, _L)

    for c in range(_NCHUNK):
        cu = pltpu.async_copy(
            utab_hbm.at[uidx_v.at[pl.ds(c * _CHUNK, _CHUNK)]], urows_v, sem_u)
        ci = pltpu.async_copy(
            itab_hbm.at[iidx_v.at[pl.ds(c * _CHUNK, _CHUNK)]], irows_v, sem_i)
        cu.wait()
        ci.wait()

        def blk(b, _):
            r0 = b * _ROWS_PER_BLK
            for k in range(_ROWS_PER_BLK):
                acc = (urows_v[r0 + k, pl.ds(0, _L)]
                       * irows_v[r0 + k, pl.ds(0, _L)])
                for d in range(1, EMBED_DIM // _L):
                    acc = acc + (urows_v[r0 + k, pl.ds(d * _L, _L)]
                                 * irows_v[r0 + k, pl.ds(d * _L, _L)])
                part_v[pl.ds(k * _L, _L)] = acc
            rowbase = lane * _L
            total = plsc.load_gather(part_v, [rowbase])
            for j in range(1, _L):
                total = total + plsc.load_gather(part_v, [rowbase + j])
            out_v[pl.ds(c * _CHUNK + r0, _L)] = 1.0 / (1.0 + jnp.exp(-total))
            return ()

        lax.fori_loop(0, _NBLK, blk, (), unroll=False)

    pltpu.sync_copy(out_v, out_hbm.at[pl.ds(base, _BPW)])


@jax.jit
def kernel(user_ids, item_ids, user_table, item_table):
    utab_p = _relayout(user_table.T)
    itab_p = _relayout(item_table.T)
    mesh = plsc.VectorSubcoreMesh(core_axis_name="c", subcore_axis_name="s")
    run = pl.kernel(
        _sc_body,
        out_type=jax.ShapeDtypeStruct((BATCH,), jnp.float32),
        mesh=mesh,
        scratch_types=[
            pltpu.VMEM((_BPW,), jnp.int32),
            pltpu.VMEM((_BPW,), jnp.int32),
            pltpu.VMEM((_CHUNK, _ROW), jnp.float32),
            pltpu.VMEM((_CHUNK, _ROW), jnp.float32),
            pltpu.VMEM((_BPW,), jnp.float32),
            pltpu.VMEM((_L * _L,), jnp.float32),
            pltpu.SemaphoreType.DMA,
            pltpu.SemaphoreType.DMA,
        ],
        compiler_params=pltpu.CompilerParams(
            needs_layout_passes=False, use_tc_tiling_on_sc=True),
    )
    return run(user_ids.astype(jnp.int32), item_ids.astype(jnp.int32),
               utab_p, itab_p)


# speedup vs baseline: 3.6305x; 1.3130x over previous
import jax
import jax.numpy as jnp
from jax import lax
from jax.experimental import pallas as pl
from jax.experimental.pallas import tpu as pltpu
from jax.experimental.pallas import tpu_sc as plsc

NUM_USERS = 100000
NUM_ITEMS = 100000
EMBED_DIM = 64
BATCH = 16384

_info = plsc.get_sparse_core_info()
_NC, _NS, _L = _info.num_cores, _info.num_subcores, _info.num_lanes
_NW = _NC * _NS
_BPW = BATCH // _NW
_CHUNK = 256
_NCHUNK = _BPW // _CHUNK
_ROWS_PER_BLK = _L
_NBLK = _CHUNK // _ROWS_PER_BLK
_ROW = 2 * EMBED_DIM

_LW = 8192
_TGRID = (NUM_USERS + _LW - 1) // _LW


def _tc_pack_body(xu_ref, xi_ref, y_ref):
    eye = jnp.eye(EMBED_DIM, dtype=jnp.float32)
    tu = jax.lax.dot_general(xu_ref[...], eye, (((0,), (0,)), ((), ())),
                             preferred_element_type=jnp.float32)
    ti = jax.lax.dot_general(xi_ref[...], eye, (((0,), (0,)), ((), ())),
                             preferred_element_type=jnp.float32)
    y_ref[...] = jnp.concatenate([tu, ti], axis=1)


def _pack(utab_t, itab_t):
    return pl.pallas_call(
        _tc_pack_body,
        grid=(_TGRID,),
        in_specs=[pl.BlockSpec((EMBED_DIM, _LW), lambda i: (0, i)),
                  pl.BlockSpec((EMBED_DIM, _LW), lambda i: (0, i))],
        out_specs=pl.BlockSpec((_LW, _ROW), lambda i: (i, 0)),
        out_shape=jax.ShapeDtypeStruct((NUM_USERS, _ROW), jnp.float32),
    )(utab_t, itab_t)


def _sc_body(uids_hbm, iids_hbm, tab_hbm, out_hbm,
             uidx_v, iidx_v, urows_v, irows_v, out_v, part_v, sem_u, sem_i):
    wid = lax.axis_index("s") * _NC + lax.axis_index("c")
    base = wid * _BPW

    pltpu.sync_copy(uids_hbm.at[pl.ds(base, _BPW)], uidx_v)
    pltpu.sync_copy(iids_hbm.at[pl.ds(base, _BPW)], iidx_v)

    lane = lax.iota(jnp.int32, _L)

    for c in range(_NCHUNK):
        cu = pltpu.async_copy(
            tab_hbm.at[uidx_v.at[pl.ds(c * _CHUNK, _CHUNK)]], urows_v, sem_u)
        ci = pltpu.async_copy(
            tab_hbm.at[iidx_v.at[pl.ds(c * _CHUNK, _CHUNK)]], irows_v, sem_i)
        cu.wait()
        ci.wait()

        def blk(b, _):
            r0 = b * _ROWS_PER_BLK
            for k in range(_ROWS_PER_BLK):
                acc = (urows_v[r0 + k, pl.ds(0, _L)]
                       * irows_v[r0 + k, pl.ds(EMBED_DIM, _L)])
                for d in range(1, EMBED_DIM // _L):
                    acc = acc + (urows_v[r0 + k, pl.ds(d * _L, _L)]
                                 * irows_v[r0 + k,
                                           pl.ds(EMBED_DIM + d * _L, _L)])
                part_v[pl.ds(k * _L, _L)] = acc
            rowbase = lane * _L
            total = plsc.load_gather(part_v, [rowbase])
            for j in range(1, _L):
                total = total + plsc.load_gather(part_v, [rowbase + j])
            out_v[pl.ds(c * _CHUNK + r0, _L)] = 1.0 / (1.0 + jnp.exp(-total))
            return ()

        lax.fori_loop(0, _NBLK, blk, (), unroll=False)

    pltpu.sync_copy(out_v, out_hbm.at[pl.ds(base, _BPW)])


@jax.jit
def kernel(user_ids, item_ids, user_table, item_table):
    tab_p = _pack(user_table.T, item_table.T)
    mesh = plsc.VectorSubcoreMesh(core_axis_name="c", subcore_axis_name="s")
    run = pl.kernel(
        _sc_body,
        out_type=jax.ShapeDtypeStruct((BATCH,), jnp.float32),
        mesh=mesh,
        scratch_types=[
            pltpu.VMEM((_BPW,), jnp.int32),
            pltpu.VMEM((_BPW,), jnp.int32),
            pltpu.VMEM((_CHUNK, _ROW), jnp.float32),
            pltpu.VMEM((_CHUNK, _ROW), jnp.float32),
            pltpu.VMEM((_BPW,), jnp.float32),
            pltpu.VMEM((_L * _L,), jnp.float32),
            pltpu.SemaphoreType.DMA,
            pltpu.SemaphoreType.DMA,
        ],
        compiler_params=pltpu.CompilerParams(
            needs_layout_passes=False, use_tc_tiling_on_sc=True),
    )
    return run(user_ids.astype(jnp.int32), item_ids.astype(jnp.int32), tab_p)


# speedup vs baseline: 3.7374x; 1.0294x over previous
import jax
import jax.numpy as jnp
from jax import lax
from jax.experimental import pallas as pl
from jax.experimental.pallas import tpu as pltpu
from jax.experimental.pallas import tpu_sc as plsc

NUM_USERS = 100000
NUM_ITEMS = 100000
EMBED_DIM = 64
BATCH = 16384

_info = plsc.get_sparse_core_info()
_NC, _NS, _L = _info.num_cores, _info.num_subcores, _info.num_lanes
_NW = _NC * _NS
_BPW = BATCH // _NW
_CHUNK = 128
_NCHUNK = _BPW // _CHUNK
_ROWS_PER_BLK = _L
_NBLK = _CHUNK // _ROWS_PER_BLK
_ROW = 2 * EMBED_DIM

_LW = 8192
_TGRID = (NUM_USERS + _LW - 1) // _LW


def _tc_pack_body(xu_ref, xi_ref, y_ref):
    eye = jnp.eye(EMBED_DIM, dtype=jnp.float32)
    tu = jax.lax.dot_general(xu_ref[...], eye, (((0,), (0,)), ((), ())),
                             preferred_element_type=jnp.float32)
    ti = jax.lax.dot_general(xi_ref[...], eye, (((0,), (0,)), ((), ())),
                             preferred_element_type=jnp.float32)
    y_ref[...] = jnp.concatenate([tu, ti], axis=1)


def _pack(utab_t, itab_t):
    return pl.pallas_call(
        _tc_pack_body,
        grid=(_TGRID,),
        in_specs=[pl.BlockSpec((EMBED_DIM, _LW), lambda i: (0, i)),
                  pl.BlockSpec((EMBED_DIM, _LW), lambda i: (0, i))],
        out_specs=pl.BlockSpec((_LW, _ROW), lambda i: (i, 0)),
        out_shape=jax.ShapeDtypeStruct((NUM_USERS, _ROW), jnp.float32),
    )(utab_t, itab_t)


def _sc_body(uids_hbm, iids_hbm, tab_hbm, out_hbm,
             uidx_v, iidx_v, urows0_v, urows1_v, irows0_v, irows1_v,
             out_v, part_v, sem_u0, sem_u1, sem_i0, sem_i1):
    wid = lax.axis_index("s") * _NC + lax.axis_index("c")
    base = wid * _BPW

    pltpu.sync_copy(uids_hbm.at[pl.ds(base, _BPW)], uidx_v)
    pltpu.sync_copy(iids_hbm.at[pl.ds(base, _BPW)], iidx_v)

    lane = lax.iota(jnp.int32, _L)
    ubufs, ibufs = [urows0_v, urows1_v], [irows0_v, irows1_v]
    usems, isems = [sem_u0, sem_u1], [sem_i0, sem_i1]

    def issue(c):
        p = c % 2
        return (pltpu.async_copy(
                    tab_hbm.at[uidx_v.at[pl.ds(c * _CHUNK, _CHUNK)]],
                    ubufs[p], usems[p]),
                pltpu.async_copy(
                    tab_hbm.at[iidx_v.at[pl.ds(c * _CHUNK, _CHUNK)]],
                    ibufs[p], isems[p]))

    pending = issue(0)
    for c in range(_NCHUNK):
        urows_v, irows_v = ubufs[c % 2], ibufs[c % 2]
        nxt = issue(c + 1) if c + 1 < _NCHUNK else None
        pending[0].wait()
        pending[1].wait()
        pending = nxt

        def blk(b, _):
            r0 = b * _ROWS_PER_BLK
            for k in range(_ROWS_PER_BLK):
                acc = (urows_v[r0 + k, pl.ds(0, _L)]
                       * irows_v[r0 + k, pl.ds(EMBED_DIM, _L)])
                for d in range(1, EMBED_DIM // _L):
                    acc = acc + (urows_v[r0 + k, pl.ds(d * _L, _L)]
                                 * irows_v[r0 + k,
                                           pl.ds(EMBED_DIM + d * _L, _L)])
                part_v[pl.ds(k * _L, _L)] = acc
            rowbase = lane * _L
            total = plsc.load_gather(part_v, [rowbase])
            for j in range(1, _L):
                total = total + plsc.load_gather(part_v, [rowbase + j])
            out_v[pl.ds(c * _CHUNK + r0, _L)] = 1.0 / (1.0 + jnp.exp(-total))
            return ()

        lax.fori_loop(0, _NBLK, blk, (), unroll=False)

    pltpu.sync_copy(out_v, out_hbm.at[pl.ds(base, _BPW)])


@jax.jit
def kernel(user_ids, item_ids, user_table, item_table):
    tab_p = _pack(user_table.T, item_table.T)
    mesh = plsc.VectorSubcoreMesh(core_axis_name="c", subcore_axis_name="s")
    run = pl.kernel(
        _sc_body,
        out_type=jax.ShapeDtypeStruct((BATCH,), jnp.float32),
        mesh=mesh,
        scratch_types=[
            pltpu.VMEM((_BPW,), jnp.int32),
            pltpu.VMEM((_BPW,), jnp.int32),
            pltpu.VMEM((_CHUNK, _ROW), jnp.float32),
            pltpu.VMEM((_CHUNK, _ROW), jnp.float32),
            pltpu.VMEM((_CHUNK, _ROW), jnp.float32),
            pltpu.VMEM((_CHUNK, _ROW), jnp.float32),
            pltpu.VMEM((_BPW,), jnp.float32),
            pltpu.VMEM((_L * _L,), jnp.float32),
            pltpu.SemaphoreType.DMA,
            pltpu.SemaphoreType.DMA,
            pltpu.SemaphoreType.DMA,
            pltpu.SemaphoreType.DMA,
        ],
        compiler_params=pltpu.CompilerParams(
            needs_layout_passes=False, use_tc_tiling_on_sc=True),
    )
    return run(user_ids.astype(jnp.int32), item_ids.astype(jnp.int32), tab_p)
